# Initial kernel scaffold; baseline (speedup 1.0000x reference)
import jax
import jax.numpy as jnp
from jax.experimental import pallas as pl


def kernel(pred, qhat):
    raise NotImplementedError("write your pallas kernel here")


# speedup vs baseline: 27.7072x; 27.7072x over previous
import functools

import jax
import jax.numpy as jnp
from jax import lax
from jax.experimental import pallas as pl
from jax.experimental.pallas import tpu as pltpu

_B = 128
_V = 100000
_ROWS = 8


def _body(q_ref, pred_ref, out_ref):
    x = pred_ref[...]
    q = q_ref[0]
    r = x.shape[0]
    v = x.shape[1]

    mx = jnp.max(x, axis=1, keepdims=True)
    mx_bits = lax.bitcast_convert_type(mx, jnp.int32)

    lo0 = jnp.zeros((r, 1), jnp.int32)
    hi0 = mx_bits + 1

    def step(_, carry):
        lo, hi = carry
        mid = (lo + hi) >> 1
        t = lax.bitcast_convert_type(mid, jnp.float32)
        s = jnp.sum(jnp.where(x >= t, x, 0.0), axis=1, keepdims=True)
        feas = s <= q
        return jnp.where(feas, lo, mid + 1), jnp.where(feas, mid, hi)

    _, hi = lax.fori_loop(0, 31, step, (lo0, hi0))
    thr = lax.bitcast_convert_type(hi, jnp.float32)

    mask = x >= thr
    ii = lax.broadcasted_iota(jnp.int32, (r, v), 1)
    amax = jnp.min(jnp.where(x == mx, ii, v), axis=1, keepdims=True)
    mask = mask | (ii == amax)

    out_ref[:, :v] = mask.astype(jnp.float32)
    out_ref[:, v:] = x


@jax.jit
def kernel(pred, qhat):
    b, v = pred.shape
    grid = b // _ROWS
    qv = jnp.reshape(qhat, (1,))
    return pl.pallas_call(
        _body,
        grid=(grid,),
        in_specs=[
            pl.BlockSpec(memory_space=pltpu.SMEM),
            pl.BlockSpec((_ROWS, v), lambda i: (i, 0)),
        ],
        out_specs=pl.BlockSpec((_ROWS, 2 * v), lambda i: (i, 0)),
        out_shape=jax.ShapeDtypeStruct((b, 2 * v), jnp.float32),
    )(qv, pred)
